# baseline (speedup 1.0000x reference)
import functools

import jax
import jax.numpy as jnp
from jax.experimental import pallas as pl
from jax.experimental.pallas import tpu as pltpu

T = 2048
HIDDEN = 1024
E = 16
TOPK = 2
FFN = 4096

ROWS = T * TOPK
BLK = 256
NBLK = ROWS // BLK
NPAIR = NBLK + E - 1
FCH = 512
NFCH = FFN // FCH


def _gemm_body(meta_ref, x_ref, w1a_ref, w1b_ref, w2_ref, out_ref):
    g = pl.program_id(0)
    f = pl.program_id(1)
    pb = meta_ref[0, g]
    lo = meta_ref[2, g]
    hi = meta_ref[3, g]
    rows = pb * BLK + jax.lax.broadcasted_iota(jnp.int32, (BLK, 1), 0)
    mask = (rows >= lo) & (rows < hi)
    x = x_ref[...]
    a = jnp.dot(x, w1a_ref[0], preferred_element_type=jnp.float32)
    b = jnp.dot(x, w1b_ref[0], preferred_element_type=jnp.float32)
    inter = a * jax.nn.sigmoid(a) * b
    inter = jnp.where(mask, inter, 0.0)
    contrib = jnp.dot(inter, w2_ref[0], preferred_element_type=jnp.float32)
    prev_pb = meta_ref[0, jnp.maximum(g - 1, 0)]
    is_first = jnp.logical_and(f == 0, jnp.logical_or(g == 0, prev_pb != pb))

    @pl.when(is_first)
    def _():
        out_ref[...] = jnp.zeros_like(out_ref)

    out_ref[...] += contrib


def _grouped_gemm(meta, permuted, W1, W2):
    grid_spec = pltpu.PrefetchScalarGridSpec(
        num_scalar_prefetch=1,
        grid=(NPAIR, NFCH),
        in_specs=[
            pl.BlockSpec((BLK, HIDDEN), lambda g, f, m: (m[0, g], 0)),
            pl.BlockSpec((1, HIDDEN, FCH), lambda g, f, m: (m[1, g], 0, f)),
            pl.BlockSpec((1, HIDDEN, FCH), lambda g, f, m: (m[1, g], 0, f + NFCH)),
            pl.BlockSpec((1, FCH, HIDDEN), lambda g, f, m: (m[1, g], f, 0)),
        ],
        out_specs=pl.BlockSpec((BLK, HIDDEN), lambda g, f, m: (m[0, g], 0)),
    )
    return pl.pallas_call(
        _gemm_body,
        grid_spec=grid_spec,
        out_shape=jax.ShapeDtypeStruct((ROWS, HIDDEN), jnp.float32),
        compiler_params=pltpu.CompilerParams(
            dimension_semantics=("arbitrary", "arbitrary"),
        ),
    )(meta, permuted, W1, W1, W2)


def _pair_metadata(offsets):
    starts = offsets[:E]
    ends = offsets[1:]
    blk_start = jnp.arange(NBLK, dtype=jnp.int32) * BLK
    blk_end = blk_start + BLK
    M = (starts[None, :] < blk_end[:, None]) & (ends[None, :] > blk_start[:, None])
    M = M & (ends > starts)[None, :]
    flat = M.reshape(-1)
    order = jnp.argsort(~flat, stable=True)[:NPAIR].astype(jnp.int32)
    valid = flat[order]
    pb = order // E
    pe = order % E
    lo = jnp.where(valid, jnp.maximum(starts[pe], pb * BLK), 0)
    hi = jnp.where(valid, jnp.minimum(ends[pe], pb * BLK + BLK), 0)
    npairs = jnp.sum(flat.astype(jnp.int32))
    last_pe = pe[npairs - 1]
    pb = jnp.where(valid, pb, NBLK - 1)
    pe = jnp.where(valid, pe, last_pe)
    return jnp.stack([pb, pe, lo, hi]).astype(jnp.int32)


def kernel(hidden_states, W_qkv, W1, W2):
    mix = hidden_states @ W_qkv
    q, k, v = jnp.split(mix, 3, axis=-1)
    attn = q[:, :, None] * k[:, None, :]
    attn = jax.nn.softmax(attn, axis=-1)
    logits = jnp.sum(attn * v[:, None, :], axis=-1)
    probs = jax.nn.softmax(logits, axis=-1)
    topk_probs, topk_idx = jax.lax.top_k(probs, TOPK)
    topk_probs = topk_probs / jnp.sum(topk_probs, axis=-1, keepdims=True)

    flat_idx = topk_idx.reshape(-1)
    flat_probs = topk_probs.reshape(-1)
    glm = jnp.repeat(jnp.arange(T), TOPK)
    counts = jnp.sum(jax.nn.one_hot(flat_idx, E, dtype=jnp.int32), axis=0)
    offsets = jnp.concatenate(
        [jnp.zeros((1,), jnp.int32), jnp.cumsum(counts)]).astype(jnp.int32)
    sort_order = jnp.argsort(flat_idx, stable=True)
    permuted = hidden_states[glm[sort_order]]
    meta = _pair_metadata(offsets)

    expert_out = _grouped_gemm(meta, permuted, W1, W2)

    unperm = jnp.zeros_like(expert_out).at[sort_order].set(expert_out)
    unperm = unperm * flat_probs[:, None]
    out = jnp.zeros((T, HIDDEN), dtype=jnp.float32).at[glm].add(unperm)
    return out


# speedup vs baseline: 1.0957x; 1.0957x over previous
import functools

import jax
import jax.numpy as jnp
from jax.experimental import pallas as pl
from jax.experimental.pallas import tpu as pltpu

T = 2048
HIDDEN = 1024
E = 16
TOPK = 2
FFN = 4096

ROWS = T * TOPK
BLK = 512
NBLK = ROWS // BLK
NPAIR = NBLK + E - 1
FCH = 512
NFCH = FFN // FCH


def _gemm_body(meta_ref, x_ref, w1a_ref, w1b_ref, w2_ref, out_ref):
    g = pl.program_id(0)
    f = pl.program_id(1)
    pb = meta_ref[0, g]
    lo = meta_ref[2, g]
    hi = meta_ref[3, g]
    rows = pb * BLK + jax.lax.broadcasted_iota(jnp.int32, (BLK, 1), 0)
    mask = (rows >= lo) & (rows < hi)
    x = x_ref[...]
    a = jnp.dot(x, w1a_ref[0], preferred_element_type=jnp.float32)
    b = jnp.dot(x, w1b_ref[0], preferred_element_type=jnp.float32)
    inter = a * jax.nn.sigmoid(a) * b
    inter = jnp.where(mask, inter, 0.0)
    contrib = jnp.dot(inter, w2_ref[0], preferred_element_type=jnp.float32)
    prev_pb = meta_ref[0, jnp.maximum(g - 1, 0)]
    is_first = jnp.logical_and(f == 0, jnp.logical_or(g == 0, prev_pb != pb))

    @pl.when(is_first)
    def _():
        out_ref[...] = jnp.zeros_like(out_ref)

    out_ref[...] += contrib


def _grouped_gemm(meta, permuted, W1, W2):
    grid_spec = pltpu.PrefetchScalarGridSpec(
        num_scalar_prefetch=1,
        grid=(NPAIR, NFCH),
        in_specs=[
            pl.BlockSpec((BLK, HIDDEN), lambda g, f, m: (m[0, g], 0)),
            pl.BlockSpec((1, HIDDEN, FCH), lambda g, f, m: (m[1, g], 0, f)),
            pl.BlockSpec((1, HIDDEN, FCH), lambda g, f, m: (m[1, g], 0, f + NFCH)),
            pl.BlockSpec((1, FCH, HIDDEN), lambda g, f, m: (m[1, g], f, 0)),
        ],
        out_specs=pl.BlockSpec((BLK, HIDDEN), lambda g, f, m: (m[0, g], 0)),
    )
    return pl.pallas_call(
        _gemm_body,
        grid_spec=grid_spec,
        out_shape=jax.ShapeDtypeStruct((ROWS, HIDDEN), jnp.float32),
        compiler_params=pltpu.CompilerParams(
            dimension_semantics=("arbitrary", "arbitrary"),
        ),
    )(meta, permuted, W1, W1, W2)


def _pair_metadata(offsets):
    starts = offsets[:E]
    ends = offsets[1:]
    blk_start = jnp.arange(NBLK, dtype=jnp.int32) * BLK
    blk_end = blk_start + BLK
    M = (starts[None, :] < blk_end[:, None]) & (ends[None, :] > blk_start[:, None])
    M = M & (ends > starts)[None, :]
    flat = M.reshape(-1)
    order = jnp.argsort(~flat, stable=True)[:NPAIR].astype(jnp.int32)
    valid = flat[order]
    pb = order // E
    pe = order % E
    lo = jnp.where(valid, jnp.maximum(starts[pe], pb * BLK), 0)
    hi = jnp.where(valid, jnp.minimum(ends[pe], pb * BLK + BLK), 0)
    npairs = jnp.sum(flat.astype(jnp.int32))
    last_pe = pe[npairs - 1]
    pb = jnp.where(valid, pb, NBLK - 1)
    pe = jnp.where(valid, pe, last_pe)
    return jnp.stack([pb, pe, lo, hi]).astype(jnp.int32)


def kernel(hidden_states, W_qkv, W1, W2):
    mix = hidden_states @ W_qkv
    q, k, v = jnp.split(mix, 3, axis=-1)
    attn = q[:, :, None] * k[:, None, :]
    attn = jax.nn.softmax(attn, axis=-1)
    logits = jnp.sum(attn * v[:, None, :], axis=-1)
    probs = jax.nn.softmax(logits, axis=-1)
    topk_probs, topk_idx = jax.lax.top_k(probs, TOPK)
    topk_probs = topk_probs / jnp.sum(topk_probs, axis=-1, keepdims=True)

    flat_idx = topk_idx.reshape(-1)
    flat_probs = topk_probs.reshape(-1)
    glm = jnp.repeat(jnp.arange(T), TOPK)
    counts = jnp.sum(jax.nn.one_hot(flat_idx, E, dtype=jnp.int32), axis=0)
    offsets = jnp.concatenate(
        [jnp.zeros((1,), jnp.int32), jnp.cumsum(counts)]).astype(jnp.int32)
    sort_order = jnp.argsort(flat_idx, stable=True)
    permuted = hidden_states[glm[sort_order]]
    meta = _pair_metadata(offsets)

    expert_out = _grouped_gemm(meta, permuted, W1, W2)

    unperm = jnp.zeros_like(expert_out).at[sort_order].set(expert_out)
    unperm = unperm * flat_probs[:, None]
    out = jnp.zeros((T, HIDDEN), dtype=jnp.float32).at[glm].add(unperm)
    return out


# speedup vs baseline: 1.1054x; 1.0089x over previous
import functools

import jax
import jax.numpy as jnp
from jax.experimental import pallas as pl
from jax.experimental.pallas import tpu as pltpu

T = 2048
HIDDEN = 1024
E = 16
TOPK = 2
FFN = 4096

ROWS = T * TOPK
BLK = 512
NBLK = ROWS // BLK
NPAIR = NBLK + E - 1
FCH = 512
NFCH = FFN // FCH


def _gemm_body(meta_ref, x_ref, w1a_ref, w1b_ref, w2_ref, out_ref):
    g = pl.program_id(0)
    f = pl.program_id(1)
    pb = meta_ref[0, g]
    lo = meta_ref[2, g]
    hi = meta_ref[3, g]
    rows = pb * BLK + jax.lax.broadcasted_iota(jnp.int32, (BLK, 1), 0)
    mask = (rows >= lo) & (rows < hi)
    x = x_ref[...].astype(jnp.bfloat16)
    w1a = w1a_ref[0].astype(jnp.bfloat16)
    w1b = w1b_ref[0].astype(jnp.bfloat16)
    a = jnp.dot(x, w1a, preferred_element_type=jnp.float32)
    b = jnp.dot(x, w1b, preferred_element_type=jnp.float32)
    inter = a * jax.nn.sigmoid(a) * b
    inter = jnp.where(mask, inter, 0.0)
    contrib = jnp.dot(inter.astype(jnp.bfloat16),
                      w2_ref[0].astype(jnp.bfloat16),
                      preferred_element_type=jnp.float32)
    prev_pb = meta_ref[0, jnp.maximum(g - 1, 0)]
    is_first = jnp.logical_and(f == 0, jnp.logical_or(g == 0, prev_pb != pb))

    @pl.when(is_first)
    def _():
        out_ref[...] = jnp.zeros_like(out_ref)

    out_ref[...] += contrib


def _grouped_gemm(meta, permuted, W1, W2):
    grid_spec = pltpu.PrefetchScalarGridSpec(
        num_scalar_prefetch=1,
        grid=(NPAIR, NFCH),
        in_specs=[
            pl.BlockSpec((BLK, HIDDEN), lambda g, f, m: (m[0, g], 0)),
            pl.BlockSpec((1, HIDDEN, FCH), lambda g, f, m: (m[1, g], 0, f)),
            pl.BlockSpec((1, HIDDEN, FCH), lambda g, f, m: (m[1, g], 0, f + NFCH)),
            pl.BlockSpec((1, FCH, HIDDEN), lambda g, f, m: (m[1, g], f, 0)),
        ],
        out_specs=pl.BlockSpec((BLK, HIDDEN), lambda g, f, m: (m[0, g], 0)),
    )
    return pl.pallas_call(
        _gemm_body,
        grid_spec=grid_spec,
        out_shape=jax.ShapeDtypeStruct((ROWS, HIDDEN), jnp.float32),
        compiler_params=pltpu.CompilerParams(
            dimension_semantics=("arbitrary", "arbitrary"),
        ),
    )(meta, permuted, W1, W1, W2)


def _pair_metadata(offsets):
    starts = offsets[:E]
    ends = offsets[1:]
    blk_start = jnp.arange(NBLK, dtype=jnp.int32) * BLK
    blk_end = blk_start + BLK
    M = (starts[None, :] < blk_end[:, None]) & (ends[None, :] > blk_start[:, None])
    M = M & (ends > starts)[None, :]
    flat = M.reshape(-1)
    order = jnp.argsort(~flat, stable=True)[:NPAIR].astype(jnp.int32)
    valid = flat[order]
    pb = order // E
    pe = order % E
    lo = jnp.where(valid, jnp.maximum(starts[pe], pb * BLK), 0)
    hi = jnp.where(valid, jnp.minimum(ends[pe], pb * BLK + BLK), 0)
    npairs = jnp.sum(flat.astype(jnp.int32))
    last_pe = pe[npairs - 1]
    pb = jnp.where(valid, pb, NBLK - 1)
    pe = jnp.where(valid, pe, last_pe)
    return jnp.stack([pb, pe, lo, hi]).astype(jnp.int32)


def kernel(hidden_states, W_qkv, W1, W2):
    mix = hidden_states @ W_qkv
    q, k, v = jnp.split(mix, 3, axis=-1)
    attn = q[:, :, None] * k[:, None, :]
    attn = jax.nn.softmax(attn, axis=-1)
    logits = jnp.sum(attn * v[:, None, :], axis=-1)
    probs = jax.nn.softmax(logits, axis=-1)
    topk_probs, topk_idx = jax.lax.top_k(probs, TOPK)
    topk_probs = topk_probs / jnp.sum(topk_probs, axis=-1, keepdims=True)

    flat_idx = topk_idx.reshape(-1)
    flat_probs = topk_probs.reshape(-1)
    glm = jnp.repeat(jnp.arange(T), TOPK)
    counts = jnp.sum(jax.nn.one_hot(flat_idx, E, dtype=jnp.int32), axis=0)
    offsets = jnp.concatenate(
        [jnp.zeros((1,), jnp.int32), jnp.cumsum(counts)]).astype(jnp.int32)
    sort_order = jnp.argsort(flat_idx, stable=True)
    permuted = hidden_states[glm[sort_order]]
    meta = _pair_metadata(offsets)

    expert_out = _grouped_gemm(meta, permuted, W1, W2)

    unperm = jnp.zeros_like(expert_out).at[sort_order].set(expert_out)
    unperm = unperm * flat_probs[:, None]
    out = jnp.zeros((T, HIDDEN), dtype=jnp.float32).at[glm].add(unperm)
    return out


# speedup vs baseline: 1.1214x; 1.0145x over previous
import functools

import jax
import jax.numpy as jnp
from jax.experimental import pallas as pl
from jax.experimental.pallas import tpu as pltpu

T = 2048
HIDDEN = 1024
E = 16
TOPK = 2
FFN = 4096

ROWS = T * TOPK
BLK = 128
NBLK = ROWS // BLK
NPAIR = NBLK + E - 1
FCH = 512
NFCH = FFN // FCH


def _ab_body(meta_ref, x_ref, w1h_ref, ab_ref):
    x = x_ref[...]
    ab_ref[0] = jnp.dot(x, w1h_ref[0],
                        preferred_element_type=jnp.float32).astype(jnp.bfloat16)


def _c2_body(meta_ref, a_ref, b_ref, w2_ref, out_ref):
    g = pl.program_id(0)
    pb = meta_ref[0, g]
    lo = meta_ref[2, g]
    hi = meta_ref[3, g]
    rows = pb * BLK + jax.lax.broadcasted_iota(jnp.int32, (BLK, 1), 0)
    mask = (rows >= lo) & (rows < hi)
    a = a_ref[0].astype(jnp.float32)
    b = b_ref[0].astype(jnp.float32)
    inter = a * jax.nn.sigmoid(a) * b
    inter = jnp.where(mask, inter, 0.0)
    contrib = jnp.dot(inter, w2_ref[0], preferred_element_type=jnp.float32)
    prev_pb = meta_ref[0, jnp.maximum(g - 1, 0)]
    is_first = jnp.logical_or(g == 0, prev_pb != pb)

    @pl.when(is_first)
    def _():
        out_ref[...] = jnp.zeros_like(out_ref)

    out_ref[...] += contrib


def _grouped_gemm(meta, permuted, W1, W2):
    ab = pl.pallas_call(
        _ab_body,
        grid_spec=pltpu.PrefetchScalarGridSpec(
            num_scalar_prefetch=1,
            grid=(2 * NPAIR,),
            in_specs=[
                pl.BlockSpec((BLK, HIDDEN), lambda s, m: (m[0, s % NPAIR], 0)),
                pl.BlockSpec((1, HIDDEN, FFN),
                             lambda s, m: (m[1, s % NPAIR], 0, s // NPAIR)),
            ],
            out_specs=pl.BlockSpec((1, BLK, FFN), lambda s, m: (s, 0, 0)),
        ),
        out_shape=jax.ShapeDtypeStruct((2 * NPAIR, BLK, FFN), jnp.bfloat16),
        compiler_params=pltpu.CompilerParams(
            dimension_semantics=("arbitrary",),
        ),
    )(meta, permuted, W1)

    return pl.pallas_call(
        _c2_body,
        grid_spec=pltpu.PrefetchScalarGridSpec(
            num_scalar_prefetch=1,
            grid=(NPAIR,),
            in_specs=[
                pl.BlockSpec((1, BLK, FFN), lambda g, m: (g, 0, 0)),
                pl.BlockSpec((1, BLK, FFN), lambda g, m: (g + NPAIR, 0, 0)),
                pl.BlockSpec((1, FFN, HIDDEN), lambda g, m: (m[1, g], 0, 0)),
            ],
            out_specs=pl.BlockSpec((BLK, HIDDEN), lambda g, m: (m[0, g], 0)),
        ),
        out_shape=jax.ShapeDtypeStruct((ROWS, HIDDEN), jnp.float32),
        compiler_params=pltpu.CompilerParams(
            dimension_semantics=("arbitrary",),
        ),
    )(meta, ab, ab, W2)


def _pair_metadata(offsets):
    starts = offsets[:E]
    ends = offsets[1:]
    blk_start = jnp.arange(NBLK, dtype=jnp.int32) * BLK
    blk_end = blk_start + BLK
    M = (starts[None, :] < blk_end[:, None]) & (ends[None, :] > blk_start[:, None])
    M = M & (ends > starts)[None, :]
    flat = M.reshape(-1)
    order = jnp.argsort(~flat, stable=True)[:NPAIR].astype(jnp.int32)
    valid = flat[order]
    pb = order // E
    pe = order % E
    lo = jnp.where(valid, jnp.maximum(starts[pe], pb * BLK), 0)
    hi = jnp.where(valid, jnp.minimum(ends[pe], pb * BLK + BLK), 0)
    npairs = jnp.sum(flat.astype(jnp.int32))
    last_pe = pe[npairs - 1]
    pb = jnp.where(valid, pb, NBLK - 1)
    pe = jnp.where(valid, pe, last_pe)
    return jnp.stack([pb, pe, lo, hi]).astype(jnp.int32)


def kernel(hidden_states, W_qkv, W1, W2):
    mix = hidden_states @ W_qkv
    q, k, v = jnp.split(mix, 3, axis=-1)
    attn = q[:, :, None] * k[:, None, :]
    attn = jax.nn.softmax(attn, axis=-1)
    logits = jnp.sum(attn * v[:, None, :], axis=-1)
    probs = jax.nn.softmax(logits, axis=-1)
    topk_probs, topk_idx = jax.lax.top_k(probs, TOPK)
    topk_probs = topk_probs / jnp.sum(topk_probs, axis=-1, keepdims=True)

    flat_idx = topk_idx.reshape(-1)
    flat_probs = topk_probs.reshape(-1)
    glm = jnp.repeat(jnp.arange(T), TOPK)
    counts = jnp.sum(jax.nn.one_hot(flat_idx, E, dtype=jnp.int32), axis=0)
    offsets = jnp.concatenate(
        [jnp.zeros((1,), jnp.int32), jnp.cumsum(counts)]).astype(jnp.int32)
    sort_order = jnp.argsort(flat_idx, stable=True)
    permuted = hidden_states[glm[sort_order]]
    meta = _pair_metadata(offsets)

    expert_out = _grouped_gemm(meta, permuted, W1, W2)

    unperm = jnp.zeros_like(expert_out).at[sort_order].set(expert_out)
    unperm = unperm * flat_probs[:, None]
    out = jnp.zeros((T, HIDDEN), dtype=jnp.float32).at[glm].add(unperm)
    return out


# speedup vs baseline: 1.1680x; 1.0415x over previous
import functools

import jax
import jax.numpy as jnp
from jax import lax
from jax.experimental import pallas as pl
from jax.experimental.pallas import tpu as pltpu
from jax.experimental.pallas import tpu_sc as plsc

T = 2048
HIDDEN = 1024
E = 16
TOPK = 2
FFN = 4096

ROWS = T * TOPK
BLK = 128
NBLK = ROWS // BLK
NPAIR = NBLK + E - 1
FCH = 512
NFCH = FFN // FCH


def _ab_body(meta_ref, x_ref, w1h_ref, ab_ref):
    x = x_ref[...]
    ab_ref[0] = jnp.dot(x, w1h_ref[0],
                        preferred_element_type=jnp.float32).astype(jnp.bfloat16)


def _c2_body(meta_ref, a_ref, b_ref, w2_ref, out_ref):
    g = pl.program_id(0)
    pb = meta_ref[0, g]
    lo = meta_ref[2, g]
    hi = meta_ref[3, g]
    rows = pb * BLK + jax.lax.broadcasted_iota(jnp.int32, (BLK, 1), 0)
    mask = (rows >= lo) & (rows < hi)
    a = a_ref[0].astype(jnp.float32)
    b = b_ref[0].astype(jnp.float32)
    inter = a * jax.nn.sigmoid(a) * b
    inter = jnp.where(mask, inter, 0.0)
    contrib = jnp.dot(inter, w2_ref[0], preferred_element_type=jnp.float32)
    prev_pb = meta_ref[0, jnp.maximum(g - 1, 0)]
    is_first = jnp.logical_or(g == 0, prev_pb != pb)

    @pl.when(is_first)
    def _():
        out_ref[...] = jnp.zeros_like(out_ref)

    out_ref[...] += contrib


def _grouped_gemm(meta, permuted, W1, W2):
    ab = pl.pallas_call(
        _ab_body,
        grid_spec=pltpu.PrefetchScalarGridSpec(
            num_scalar_prefetch=1,
            grid=(2 * NPAIR,),
            in_specs=[
                pl.BlockSpec((BLK, HIDDEN), lambda s, m: (m[0, s % NPAIR], 0)),
                pl.BlockSpec((1, HIDDEN, FFN),
                             lambda s, m: (m[1, s % NPAIR], 0, s // NPAIR)),
            ],
            out_specs=pl.BlockSpec((1, BLK, FFN), lambda s, m: (s, 0, 0)),
        ),
        out_shape=jax.ShapeDtypeStruct((2 * NPAIR, BLK, FFN), jnp.bfloat16),
        compiler_params=pltpu.CompilerParams(
            dimension_semantics=("arbitrary",),
        ),
    )(meta, permuted, W1)

    return pl.pallas_call(
        _c2_body,
        grid_spec=pltpu.PrefetchScalarGridSpec(
            num_scalar_prefetch=1,
            grid=(NPAIR,),
            in_specs=[
                pl.BlockSpec((1, BLK, FFN), lambda g, m: (g, 0, 0)),
                pl.BlockSpec((1, BLK, FFN), lambda g, m: (g + NPAIR, 0, 0)),
                pl.BlockSpec((1, FFN, HIDDEN), lambda g, m: (m[1, g], 0, 0)),
            ],
            out_specs=pl.BlockSpec((BLK, HIDDEN), lambda g, m: (m[0, g], 0)),
        ),
        out_shape=jax.ShapeDtypeStruct((ROWS, HIDDEN), jnp.float32),
        compiler_params=pltpu.CompilerParams(
            dimension_semantics=("arbitrary",),
        ),
    )(meta, ab, ab, W2)


RBLK = 512
NRBLK = T // RBLK


def _router_body(x_ref, wq_ref, idx2_ref, rloc2_ref, probs2_ref, cnt_ref):
    x = x_ref[...]
    mix = jnp.dot(x, wq_ref[...], preferred_element_type=jnp.float32)
    q = mix[:, :E]
    k = mix[:, E:2 * E]
    v = mix[:, 2 * E:3 * E]
    a3 = q[:, :, None] * k[:, None, :]
    m = jnp.max(a3, axis=-1, keepdims=True)
    ex = jnp.exp(a3 - m)
    attn = ex / jnp.sum(ex, axis=-1, keepdims=True)
    logits = jnp.sum(attn * v[:, None, :], axis=-1)
    lm = jnp.max(logits, axis=-1, keepdims=True)
    el = jnp.exp(logits - lm)
    probs = el / jnp.sum(el, axis=-1, keepdims=True)
    iota = lax.broadcasted_iota(jnp.int32, (RBLK, E), 1)
    m1 = jnp.max(probs, axis=-1, keepdims=True)
    i1 = jnp.min(jnp.where(probs == m1, iota, E), axis=-1)
    masked = jnp.where(iota == i1[:, None], -1.0, probs)
    m2 = jnp.max(masked, axis=-1, keepdims=True)
    i2 = jnp.min(jnp.where(masked == m2, iota, E), axis=-1)
    psum = m1 + m2
    p1 = (m1 / psum)[:, 0]
    p2 = (m2 / psum)[:, 0]
    oh1 = (iota == i1[:, None]).astype(jnp.float32)
    oh2 = (iota == i2[:, None]).astype(jnp.float32)
    c = oh1 + oh2
    ri = lax.broadcasted_iota(jnp.int32, (RBLK, RBLK), 0)
    ci = lax.broadcasted_iota(jnp.int32, (RBLK, RBLK), 1)
    ltri = (ri > ci).astype(jnp.float32)
    s = jnp.dot(ltri, c, preferred_element_type=jnp.float32)
    r1 = jnp.sum(s * oh1, axis=-1)
    r2 = jnp.sum(s * oh2, axis=-1)
    idx2_ref[...] = jnp.concatenate(
        [i1.reshape(1, RBLK), i2.reshape(1, RBLK)], axis=0)
    rloc2_ref[...] = jnp.concatenate(
        [r1.reshape(1, RBLK), r2.reshape(1, RBLK)], axis=0).astype(jnp.int32)
    probs2_ref[...] = jnp.concatenate(
        [p1.reshape(1, RBLK), p2.reshape(1, RBLK)], axis=0)
    tot = jnp.sum(c, axis=0)
    cnt_ref[...] = jnp.broadcast_to(tot.reshape(1, 1, E), (1, 8, E))


def _router(hidden_states, W_qkv):
    return pl.pallas_call(
        _router_body,
        grid=(NRBLK,),
        in_specs=[
            pl.BlockSpec((RBLK, HIDDEN), lambda b: (b, 0)),
            pl.BlockSpec((HIDDEN, 3 * E), lambda b: (0, 0)),
        ],
        out_specs=[
            pl.BlockSpec((2, RBLK), lambda b: (0, b)),
            pl.BlockSpec((2, RBLK), lambda b: (0, b)),
            pl.BlockSpec((2, RBLK), lambda b: (0, b)),
            pl.BlockSpec((1, 8, E), lambda b: (b, 0, 0)),
        ],
        out_shape=[
            jax.ShapeDtypeStruct((2, T), jnp.int32),
            jax.ShapeDtypeStruct((2, T), jnp.int32),
            jax.ShapeDtypeStruct((2, T), jnp.float32),
            jax.ShapeDtypeStruct((NRBLK, 8, E), jnp.float32),
        ],
        compiler_params=pltpu.CompilerParams(
            dimension_semantics=("arbitrary",),
        ),
    )(hidden_states, W_qkv)


NW = 32
TPW = T // NW


def _finalize_body(cnt_ref, idx_ref, rl_ref, pr_ref,
                   pos0_ref, pos1_ref, prb0_ref, prb1_ref):
    b = pl.program_id(0)
    cnt = cnt_ref[:, 0, :]
    totals = jnp.sum(cnt, axis=0)
    ce = lax.broadcasted_iota(jnp.int32, (E, E), 0)
    ce2 = lax.broadcasted_iota(jnp.int32, (E, E), 1)
    base16 = jnp.sum(jnp.where(ce < ce2, totals[:, None], 0.0),
                     axis=0).reshape(1, E)
    bi = lax.broadcasted_iota(jnp.int32, (NRBLK, 1), 0)
    prefix_b = jnp.sum(jnp.where(bi < b, cnt, 0.0), axis=0,
                       keepdims=True)
    table_b = base16 + prefix_b
    iota = lax.broadcasted_iota(jnp.int32, (RBLK, E), 1)
    i1 = idx_ref[0][:, None]
    i2 = idx_ref[1][:, None]
    sel1 = jnp.sum(jnp.where(iota == i1, table_b, 0.0), axis=-1)
    sel2 = jnp.sum(jnp.where(iota == i2, table_b, 0.0), axis=-1)
    pos0_ref[...] = sel1.astype(jnp.int32) + rl_ref[0]
    pos1_ref[...] = sel2.astype(jnp.int32) + rl_ref[1]
    prb0_ref[...] = jnp.broadcast_to(pr_ref[0][:, None], (RBLK, E))
    prb1_ref[...] = jnp.broadcast_to(pr_ref[1][:, None], (RBLK, E))


def _finalize(cnt3, idx2, rloc2, probs2):
    return pl.pallas_call(
        _finalize_body,
        grid=(NRBLK,),
        in_specs=[
            pl.BlockSpec((NRBLK, 8, E), lambda b: (0, 0, 0)),
            pl.BlockSpec((2, RBLK), lambda b: (0, b)),
            pl.BlockSpec((2, RBLK), lambda b: (0, b)),
            pl.BlockSpec((2, RBLK), lambda b: (0, b)),
        ],
        out_specs=[
            pl.BlockSpec((RBLK,), lambda b: (b,)),
            pl.BlockSpec((RBLK,), lambda b: (b,)),
            pl.BlockSpec((RBLK, E), lambda b: (b, 0)),
            pl.BlockSpec((RBLK, E), lambda b: (b, 0)),
        ],
        out_shape=[
            jax.ShapeDtypeStruct((T,), jnp.int32),
            jax.ShapeDtypeStruct((T,), jnp.int32),
            jax.ShapeDtypeStruct((T, E), jnp.float32),
            jax.ShapeDtypeStruct((T, E), jnp.float32),
        ],
        compiler_params=pltpu.CompilerParams(
            dimension_semantics=("arbitrary",),
        ),
    )(cnt3, idx2, rloc2, probs2)


def _make_gather_kernel():
    mesh = plsc.VectorSubcoreMesh(core_axis_name="c", subcore_axis_name="s")

    @functools.partial(
        pl.kernel, mesh=mesh,
        out_type=jax.ShapeDtypeStruct((ROWS, HIDDEN), jnp.float32),
        scratch_types=[
            pltpu.VMEM((2, 32), jnp.int32),
            pltpu.VMEM((2, 32), jnp.int32),
            pltpu.VMEM((TPW, HIDDEN), jnp.float32),
            pltpu.SemaphoreType.DMA,
        ],
    )
    def k(hid_hbm, pos0_hbm, pos1_hbm, out_hbm, p0_v, p1_v, rows_v, sem):
        wid = lax.axis_index("s") * 2 + lax.axis_index("c")
        base = wid * TPW
        pltpu.sync_copy(pos0_hbm.at[pl.ds(base, 32)], p0_v.at[0])
        pltpu.sync_copy(pos0_hbm.at[pl.ds(base + 32, 32)], p0_v.at[1])
        pltpu.sync_copy(pos1_hbm.at[pl.ds(base, 32)], p1_v.at[0])
        pltpu.sync_copy(pos1_hbm.at[pl.ds(base + 32, 32)], p1_v.at[1])
        pltpu.sync_copy(hid_hbm.at[pl.ds(base, TPW)], rows_v)
        for h in range(2):
            src = rows_v.at[pl.ds(h * 32, 32)]
            pltpu.async_copy(src, out_hbm.at[p0_v.at[h]], sem).wait()
            pltpu.async_copy(src, out_hbm.at[p1_v.at[h]], sem).wait()

    return k


def _make_combine_kernel():
    mesh = plsc.VectorSubcoreMesh(core_axis_name="c", subcore_axis_name="s")

    @functools.partial(
        pl.kernel, mesh=mesh,
        out_type=jax.ShapeDtypeStruct((T, HIDDEN), jnp.float32),
        scratch_types=[
            pltpu.VMEM((2, 32), jnp.int32),
            pltpu.VMEM((2, 32), jnp.int32),
            pltpu.VMEM((TPW, E), jnp.float32),
            pltpu.VMEM((TPW, E), jnp.float32),
            pltpu.VMEM((32, HIDDEN), jnp.float32),
            pltpu.VMEM((32, HIDDEN), jnp.float32),
            pltpu.VMEM((32, HIDDEN), jnp.float32),
            pltpu.SemaphoreType.DMA,
        ],
    )
    def k(eo_hbm, pos0_hbm, pos1_hbm, prb0_hbm, prb1_hbm, out_hbm,
          p0_v, p1_v, pr0_v, pr1_v, bufe, bufo, outb, sem):
        wid = lax.axis_index("s") * 2 + lax.axis_index("c")
        base = wid * TPW
        pltpu.sync_copy(pos0_hbm.at[pl.ds(base, 32)], p0_v.at[0])
        pltpu.sync_copy(pos0_hbm.at[pl.ds(base + 32, 32)], p0_v.at[1])
        pltpu.sync_copy(pos1_hbm.at[pl.ds(base, 32)], p1_v.at[0])
        pltpu.sync_copy(pos1_hbm.at[pl.ds(base + 32, 32)], p1_v.at[1])
        pltpu.sync_copy(prb0_hbm.at[pl.ds(base, TPW)], pr0_v)
        pltpu.sync_copy(prb1_hbm.at[pl.ds(base, TPW)], pr1_v)
        for h in range(2):
            pltpu.async_copy(eo_hbm.at[p0_v.at[h]], bufe, sem).wait()
            pltpu.async_copy(eo_hbm.at[p1_v.at[h]], bufo, sem).wait()

            def body(j, _):
                pv0 = pr0_v[h * 32 + j, :]
                pv1 = pr1_v[h * 32 + j, :]
                for c in range(HIDDEN // E):
                    ev = bufe[j, pl.ds(c * E, E)]
                    ov = bufo[j, pl.ds(c * E, E)]
                    outb[j, pl.ds(c * E, E)] = pv0 * ev + pv1 * ov
                return 0

            lax.fori_loop(0, 32, body, 0)
            pltpu.sync_copy(outb, out_hbm.at[pl.ds(base + h * 32, 32)])

    return k


def _pair_metadata(offsets):
    starts = offsets[:E]
    ends = offsets[1:]
    blk_start = jnp.arange(NBLK, dtype=jnp.int32) * BLK
    blk_end = blk_start + BLK
    M = (starts[None, :] < blk_end[:, None]) & (ends[None, :] > blk_start[:, None])
    M = M & (ends > starts)[None, :]
    flat = M.reshape(-1)
    order = jnp.argsort(~flat, stable=True)[:NPAIR].astype(jnp.int32)
    valid = flat[order]
    pb = order // E
    pe = order % E
    lo = jnp.where(valid, jnp.maximum(starts[pe], pb * BLK), 0)
    hi = jnp.where(valid, jnp.minimum(ends[pe], pb * BLK + BLK), 0)
    npairs = jnp.sum(flat.astype(jnp.int32))
    last_pe = pe[npairs - 1]
    pb = jnp.where(valid, pb, NBLK - 1)
    pe = jnp.where(valid, pe, last_pe)
    return jnp.stack([pb, pe, lo, hi]).astype(jnp.int32)


def kernel(hidden_states, W_qkv, W1, W2):
    idx2, rloc2, probs2, cnt3 = _router(hidden_states, W_qkv)

    pos0, pos1, prb0, prb1 = _finalize(cnt3, idx2, rloc2, probs2)

    cnt = cnt3[:, 0, :].astype(jnp.int32)
    totals = jnp.sum(cnt, axis=0)
    offsets = jnp.concatenate(
        [jnp.zeros((1,), jnp.int32),
         jnp.cumsum(totals)]).astype(jnp.int32)
    meta = _pair_metadata(offsets)

    permuted = _make_gather_kernel()(hidden_states, pos0, pos1)

    expert_out = _grouped_gemm(meta, permuted, W1, W2)

    out = _make_combine_kernel()(expert_out, pos0, pos1, prb0, prb1)
    return out
